# baseline (speedup 1.0000x reference)
import jax
import jax.numpy as jnp
from jax.experimental import pallas as pl

_ADV_W = 0.6
_NORM_W = 0.05
_DATA_W = 0.25
_CHUNK = 256


def _loss_kernel(d_ref, fake_ref, real_ref, out_ref):
    step = pl.program_id(0)
    n_slice = 4 * 2048 * 3

    @pl.when(step == 0)
    def _init():
        adv = -jnp.sum(jnp.log(d_ref[...] + 1e-08)) / d_ref.size
        out_ref[...] = jnp.reshape(_ADV_W * adv, (1, 1))

    diff = real_ref[...] - fake_ref[...]
    ch = jax.lax.broadcasted_iota(jnp.int32, diff.shape, 2)
    w_sqrt = jnp.where(ch < 3, (_DATA_W / n_slice) ** 0.5,
                       (_NORM_W / n_slice) ** 0.5)
    t = diff * w_sqrt
    out_ref[...] += jnp.reshape(jnp.sum(t * t), (1, 1))


def kernel(D_output_fake, fake_data, real_data):
    grid = 2048 // _CHUNK
    out = pl.pallas_call(
        _loss_kernel,
        grid=(grid,),
        in_specs=[
            pl.BlockSpec((4, 1), lambda i: (0, 0)),
            pl.BlockSpec((4, _CHUNK, 6), lambda i: (0, i, 0)),
            pl.BlockSpec((4, _CHUNK, 6), lambda i: (0, i, 0)),
        ],
        out_specs=pl.BlockSpec((1, 1), lambda i: (0, 0)),
        out_shape=jax.ShapeDtypeStruct((1, 1), jnp.float32),
    )(D_output_fake, fake_data, real_data)
    return out[0, 0]


# speedup vs baseline: 1.1066x; 1.1066x over previous
import jax
import jax.numpy as jnp
from jax.experimental import pallas as pl
from jax.experimental.pallas import tpu as pltpu

_ADV_W = 0.6
_NORM_W = 0.05
_DATA_W = 0.25
_NCHUNK = 8
_CHUNK = 2048 // _NCHUNK


def _loss_kernel(d_ref, fake_hbm, real_hbm, out_ref, fake_v, real_v, sems):
    for c in range(_NCHUNK):
        sl = pl.ds(c * _CHUNK, _CHUNK)
        pltpu.make_async_copy(
            fake_hbm.at[:, sl, :], fake_v.at[:, sl, :], sems.at[2 * c]
        ).start()
        pltpu.make_async_copy(
            real_hbm.at[:, sl, :], real_v.at[:, sl, :], sems.at[2 * c + 1]
        ).start()
    for c in range(_NCHUNK):
        sl = pl.ds(c * _CHUNK, _CHUNK)
        pltpu.make_async_copy(
            fake_hbm.at[:, sl, :], fake_v.at[:, sl, :], sems.at[2 * c]
        ).wait()
        pltpu.make_async_copy(
            real_hbm.at[:, sl, :], real_v.at[:, sl, :], sems.at[2 * c + 1]
        ).wait()

    n_slice = 4 * 2048 * 3
    adv = -jnp.sum(jnp.log(d_ref[...] + 1e-08)) / d_ref.size
    diff = real_v[...] - fake_v[...]
    ch = jax.lax.broadcasted_iota(jnp.int32, diff.shape, 2)
    w_sqrt = jnp.where(ch < 3, (_DATA_W / n_slice) ** 0.5,
                       (_NORM_W / n_slice) ** 0.5)
    t = diff * w_sqrt
    out_ref[...] = jnp.reshape(_ADV_W * adv + jnp.sum(t * t), (1, 1))


def kernel(D_output_fake, fake_data, real_data):
    out = pl.pallas_call(
        _loss_kernel,
        in_specs=[
            pl.BlockSpec(memory_space=pltpu.MemorySpace.VMEM),
            pl.BlockSpec(memory_space=pltpu.MemorySpace.HBM),
            pl.BlockSpec(memory_space=pltpu.MemorySpace.HBM),
        ],
        out_specs=pl.BlockSpec(memory_space=pltpu.MemorySpace.VMEM),
        scratch_shapes=[
            pltpu.VMEM((4, 2048, 6), jnp.float32),
            pltpu.VMEM((4, 2048, 6), jnp.float32),
            pltpu.SemaphoreType.DMA((2 * _NCHUNK,)),
        ],
        out_shape=jax.ShapeDtypeStruct((1, 1), jnp.float32),
    )(D_output_fake, fake_data, real_data)
    return out[0, 0]
